# baseline (speedup 1.0000x reference)
import functools

import jax
import jax.numpy as jnp
from jax import lax
from jax.experimental import pallas as pl
from jax.experimental.pallas import tpu as pltpu
from jax.experimental.pallas import tpu_sc as plsc

_N = 10000
_E = 320000
_D = 128
_DT = 144
_LDEN = 130
_NPAD = 10016
_K = 64
_NW = 32
_NB = 3 * (-(-(_E + _N) // (_NW * _K * 3)))
_EW = _NB * _K
_EPAD = _NW * _EW
_RPT = _NPAD // 16
_RB = 2504

_f32 = jnp.float32


def _project_tail(x, w_ref, asr_ref, adr_ref, t_ref, d_ref, g_ref):
    h = jnp.dot(x, w_ref[...], preferred_element_type=_f32)
    a_s = jnp.dot(h, asr_ref[...], preferred_element_type=_f32)
    a_d = jnp.dot(h, adr_ref[...], preferred_element_type=_f32)
    ones = jnp.ones((x.shape[0], 1), _f32)
    zeros13 = jnp.zeros((x.shape[0], 13), _f32)
    t_ref[...] = jnp.concatenate([h, a_s, a_d, ones, zeros13], axis=1)
    d_ref[...] = a_d

    @pl.when(pl.program_id(0) == 0)
    def _init():
        g_ref[...] = jnp.full((1, 128), -3.0e38, _f32)

    g_ref[...] = jnp.maximum(g_ref[...], jnp.max(a_s))


def _proj1_body(x_ref, w_ref, asr_ref, adr_ref, t_ref, d_ref, g_ref):
    _project_tail(x_ref[...], w_ref, asr_ref, adr_ref, t_ref, d_ref, g_ref)


def _proj2_body(acc_ref, b_ref, w_ref, asr_ref, adr_ref, t_ref, d_ref, g_ref):
    num = acc_ref[0, :, :_D] + acc_ref[1, :, :_D]
    den = acc_ref[0, :, _LDEN:_LDEN + 1] + acc_ref[1, :, _LDEN:_LDEN + 1]
    x = jnp.maximum(num / (den + 1e-16) + b_ref[...], 0.0)
    _project_tail(x, w_ref, asr_ref, adr_ref, t_ref, d_ref, g_ref)


def _final_body(acc_ref, b_ref, o_ref):
    num = acc_ref[0, :, :_D] + acc_ref[1, :, :_D]
    den = acc_ref[0, :, _LDEN:_LDEN + 1] + acc_ref[1, :, _LDEN:_LDEN + 1]
    o_ref[...] = jnp.maximum(num / (den + 1e-16) + b_ref[...], 0.0)


_PROJ_OUT = [
    jax.ShapeDtypeStruct((_NPAD, _DT), _f32),
    jax.ShapeDtypeStruct((_NPAD, 1), _f32),
    jax.ShapeDtypeStruct((1, 128), _f32),
]
_PROJ_OUT_SPECS = [
    pl.BlockSpec((_RB, _DT), lambda i: (i, 0)),
    pl.BlockSpec((_RB, 1), lambda i: (i, 0)),
    pl.BlockSpec((1, 128), lambda i: (0, 0)),
]

_tc_project1 = pl.pallas_call(
    _proj1_body,
    grid=(_NPAD // _RB,),
    in_specs=[
        pl.BlockSpec((_RB, _D), lambda i: (i, 0)),
        pl.BlockSpec((_D, _D), lambda i: (0, 0)),
        pl.BlockSpec((_D, 1), lambda i: (0, 0)),
        pl.BlockSpec((_D, 1), lambda i: (0, 0)),
    ],
    out_specs=_PROJ_OUT_SPECS,
    out_shape=_PROJ_OUT,
)

_tc_project2 = pl.pallas_call(
    _proj2_body,
    grid=(_NPAD // _RB,),
    in_specs=[
        pl.BlockSpec((2, _RB, _DT), lambda i: (0, i, 0)),
        pl.BlockSpec((1, _D), lambda i: (0, 0)),
        pl.BlockSpec((_D, _D), lambda i: (0, 0)),
        pl.BlockSpec((_D, 1), lambda i: (0, 0)),
        pl.BlockSpec((_D, 1), lambda i: (0, 0)),
    ],
    out_specs=_PROJ_OUT_SPECS,
    out_shape=_PROJ_OUT,
)

_FB = 400
_tc_final = pl.pallas_call(
    _final_body,
    grid=(_N // _FB,),
    in_specs=[
        pl.BlockSpec((2, _FB, _DT), lambda i: (0, i, 0)),
        pl.BlockSpec((1, _D), lambda i: (0, 0)),
    ],
    out_specs=pl.BlockSpec((_FB, _D), lambda i: (i, 0)),
    out_shape=jax.ShapeDtypeStruct((_N, _D), _f32),
)


_mesh = plsc.VectorSubcoreMesh(core_axis_name="c", subcore_axis_name="s")


@functools.partial(
    pl.kernel,
    mesh=_mesh,
    compiler_params=pltpu.CompilerParams(
        needs_layout_passes=False, use_tc_tiling_on_sc=False
    ),
    out_type=[
        jax.ShapeDtypeStruct((2, _NPAD, _DT), _f32),
    ],
    scratch_types=[
        pltpu.VMEM((16,), _f32),
        [pltpu.VMEM((_K,), jnp.int32) for _ in range(3)],
        [pltpu.VMEM((_K,), jnp.int32) for _ in range(3)],
        [pltpu.VMEM((_K, _DT), _f32) for _ in range(3)],
        pltpu.VMEM((_NPAD,), _f32),
        pltpu.VMEM((_K,), _f32),
        pltpu.VMEM_SHARED((_NPAD, _DT), _f32),
        [pltpu.SemaphoreType.DMA for _ in range(3)],
        [pltpu.SemaphoreType.DMA for _ in range(3)],
    ],
)
def _sc_edge(src_hbm, dst_hbm, tail_hbm, t_hbm, d_hbm, g_hbm, acc_hbm,
             g_v, src_s, dst_s, rows_s, ad_t, ex_v, acc_s, semg, sems):
    c = lax.axis_index("c")
    s = lax.axis_index("s")
    wid = s * 2 + c

    zero16 = jnp.zeros((16,), _f32)

    def _zrow(r, carry):
        for j in range(_DT // 16):
            rows_s[0][r, pl.ds(j * 16, 16)] = zero16
        return carry

    lax.fori_loop(0, _K, _zrow, 0)

    for t in range(_RPT // _K):
        pltpu.sync_copy(rows_s[0], acc_s.at[pl.ds(s * _RPT + t * _K, _K)])
    pltpu.sync_copy(rows_s[0], acc_s.at[pl.ds(s * _RPT + _RPT - _K, _K)])

    pltpu.sync_copy(g_hbm, g_v)
    pltpu.sync_copy(d_hbm, ad_t)
    plsc.subcore_barrier()

    iota16 = lax.iota(jnp.int32, 16)
    zero16i = jnp.zeros((16,), jnp.int32)
    l_as = jnp.full((16,), _D, jnp.int32)

    def _issue(slot, b):
        off = (wid * _NB + b) * _K

        @pl.when(off < _E)
        def _real():
            pltpu.sync_copy(src_hbm.at[pl.ds(off, _K)], src_s[slot])
            pltpu.sync_copy(dst_hbm.at[pl.ds(off, _K)], dst_s[slot])

        @pl.when(off >= _E)
        def _tail():
            pltpu.sync_copy(tail_hbm.at[pl.ds(off - _E, _K)], src_s[slot])
            pltpu.sync_copy(tail_hbm.at[pl.ds(off - _E, _K)], dst_s[slot])

        pltpu.async_copy(t_hbm.at[src_s[slot]], rows_s[slot], semg[slot])

    def _wait_gather(slot):
        pltpu.make_async_copy(t_hbm.at[src_s[slot]], rows_s[slot],
                              semg[slot]).wait()

    def _wait_scatter(slot):
        pltpu.make_async_copy(rows_s[slot], acc_s.at[dst_s[slot]],
                              sems[slot]).wait()

    _issue(0, 0)
    _issue(1, 1)

    def _group(g, carry):
        for q in range(3):
            b = 3 * g + q
            rows_v = rows_s[q]
            _wait_gather(q)
            gv = g_v[...]
            for i in range(_K // 16):
                e16 = i * 16 + iota16
                d16 = dst_s[q][pl.ds(i * 16, 16)]
                a_s = plsc.load_gather(rows_v, [e16, l_as])
                a_d = plsc.load_gather(ad_t, [d16])
                t0 = a_s + a_d
                al = jnp.maximum(t0, 0.2 * t0)
                s0 = gv + a_d
                sd = jnp.maximum(s0, 0.2 * s0)
                ex_v[pl.ds(i * 16, 16)] = jnp.exp(al - sd)

            @plsc.parallel_loop(0, _K, 1, unroll=16)
            def _scale(r):
                ev = plsc.load_gather(ex_v, [jnp.full((16,), r, jnp.int32)])
                for j in range(_DT // 16):
                    rows_v[r, pl.ds(j * 16, 16)] = (
                        rows_v[r, pl.ds(j * 16, 16)] * ev
                    )
            pltpu.async_copy(rows_v, acc_s.at[dst_s[q]], sems[q], add=True)

            nxt = (q + 2) % 3

            @pl.when(b + 2 < _NB)
            def _prefetch():
                @pl.when(b >= 1)
                def _drain():
                    _wait_scatter(nxt)

                _issue(nxt, b + 2)

        return carry

    lax.fori_loop(0, _NB // 3, _group, 0)
    for q in range(3):
        _wait_scatter(q)
    plsc.subcore_barrier()

    offs = [s * _RPT + t * _K for t in range(_RPT // _K)]
    offs.append(s * _RPT + _RPT - _K)
    for i, r0 in enumerate(offs):
        pltpu.async_copy(acc_s.at[pl.ds(r0, _K)], acc_hbm.at[c, pl.ds(r0, _K)],
                         semg[i % 3])
    for i, r0 in enumerate(offs):
        pltpu.make_async_copy(acc_s.at[pl.ds(r0, _K)],
                              acc_hbm.at[c, pl.ds(r0, _K)], semg[i % 3]).wait()


def kernel(x, edge_index, W1, a_src1, a_dst1, b1,
           W2, a_src2, a_dst2, b2, W3, a_src3, a_dst3, b3):
    x_pad = jnp.pad(x, ((0, _NPAD - _N), (0, 0)))
    src = edge_index[0].astype(jnp.int32)
    dst = edge_index[1].astype(jnp.int32)
    tail = jnp.concatenate([
        jnp.arange(_N, dtype=jnp.int32),
        jnp.full((_EPAD - _E - _N,), _N, jnp.int32),
    ])

    def col(v):
        return v.reshape(_D, 1)

    def row(v):
        return v.reshape(1, _D)

    t, d, g = _tc_project1(x_pad, W1, col(a_src1), col(a_dst1))
    (acc,) = _sc_edge(src, dst, tail, t, d.reshape(-1), g[0, :16])
    t, d, g = _tc_project2(acc, row(b1), W2, col(a_src2), col(a_dst2))
    (acc,) = _sc_edge(src, dst, tail, t, d.reshape(-1), g[0, :16])
    t, d, g = _tc_project2(acc, row(b2), W3, col(a_src3), col(a_dst3))
    (acc,) = _sc_edge(src, dst, tail, t, d.reshape(-1), g[0, :16])
    return _tc_final(acc, row(b3))


# speedup vs baseline: 1.2628x; 1.2628x over previous
import functools

import jax
import jax.numpy as jnp
from jax import lax
from jax.experimental import pallas as pl
from jax.experimental.pallas import tpu as pltpu
from jax.experimental.pallas import tpu_sc as plsc

_N = 10000
_E = 320000
_D = 128
_DT = 144
_LDEN = 130
_NPAD = 10016
_K = 80
_NW = 32
_NB = 3 * (-(-(_E + _N) // (_NW * _K * 3)))
_EW = _NB * _K
_EPAD = _NW * _EW
_RPT = _NPAD // 16
_RB = 2504

_f32 = jnp.float32


def _project_tail(x, w_ref, asr_ref, adr_ref, t_ref, d_ref, g_ref):
    h = jnp.dot(x, w_ref[...], preferred_element_type=_f32)
    a_s = jnp.dot(h, asr_ref[...], preferred_element_type=_f32)
    a_d = jnp.dot(h, adr_ref[...], preferred_element_type=_f32)
    ones = jnp.ones((x.shape[0], 1), _f32)
    zeros13 = jnp.zeros((x.shape[0], 13), _f32)
    t_ref[...] = jnp.concatenate([h, a_s, a_d, ones, zeros13], axis=1)
    zeros15 = jnp.zeros((x.shape[0], 15), _f32)
    d_ref[...] = jnp.concatenate([a_d, zeros15], axis=1)

    @pl.when(pl.program_id(0) == 0)
    def _init():
        g_ref[...] = jnp.full((1, 128), -3.0e38, _f32)

    g_ref[...] = jnp.maximum(g_ref[...], jnp.max(a_s))


def _proj1_body(x_ref, w_ref, asr_ref, adr_ref, t_ref, d_ref, g_ref):
    _project_tail(x_ref[...], w_ref, asr_ref, adr_ref, t_ref, d_ref, g_ref)


def _proj2_body(acc_ref, b_ref, w_ref, asr_ref, adr_ref, t_ref, d_ref, g_ref):
    num = acc_ref[0, :, :_D] + acc_ref[1, :, :_D]
    den = acc_ref[0, :, _LDEN:_LDEN + 1] + acc_ref[1, :, _LDEN:_LDEN + 1]
    x = jnp.maximum(num / (den + 1e-16) + b_ref[...], 0.0)
    _project_tail(x, w_ref, asr_ref, adr_ref, t_ref, d_ref, g_ref)


def _final_body(acc_ref, b_ref, o_ref):
    num = acc_ref[0, :, :_D] + acc_ref[1, :, :_D]
    den = acc_ref[0, :, _LDEN:_LDEN + 1] + acc_ref[1, :, _LDEN:_LDEN + 1]
    o_ref[...] = jnp.maximum(num / (den + 1e-16) + b_ref[...], 0.0)


_PROJ_OUT = [
    jax.ShapeDtypeStruct((_NPAD, _DT), _f32),
    jax.ShapeDtypeStruct((_NPAD, 16), _f32),
    jax.ShapeDtypeStruct((1, 128), _f32),
]
_PROJ_OUT_SPECS = [
    pl.BlockSpec((_RB, _DT), lambda i: (i, 0)),
    pl.BlockSpec((_RB, 16), lambda i: (i, 0)),
    pl.BlockSpec((1, 128), lambda i: (0, 0)),
]

_tc_project1 = pl.pallas_call(
    _proj1_body,
    grid=(_NPAD // _RB,),
    in_specs=[
        pl.BlockSpec((_RB, _D), lambda i: (i, 0)),
        pl.BlockSpec((_D, _D), lambda i: (0, 0)),
        pl.BlockSpec((_D, 1), lambda i: (0, 0)),
        pl.BlockSpec((_D, 1), lambda i: (0, 0)),
    ],
    out_specs=_PROJ_OUT_SPECS,
    out_shape=_PROJ_OUT,
)

_tc_project2 = pl.pallas_call(
    _proj2_body,
    grid=(_NPAD // _RB,),
    in_specs=[
        pl.BlockSpec((2, _RB, _DT), lambda i: (0, i, 0)),
        pl.BlockSpec((1, _D), lambda i: (0, 0)),
        pl.BlockSpec((_D, _D), lambda i: (0, 0)),
        pl.BlockSpec((_D, 1), lambda i: (0, 0)),
        pl.BlockSpec((_D, 1), lambda i: (0, 0)),
    ],
    out_specs=_PROJ_OUT_SPECS,
    out_shape=_PROJ_OUT,
)

_FB = 400
_tc_final = pl.pallas_call(
    _final_body,
    grid=(_N // _FB,),
    in_specs=[
        pl.BlockSpec((2, _FB, _DT), lambda i: (0, i, 0)),
        pl.BlockSpec((1, _D), lambda i: (0, 0)),
    ],
    out_specs=pl.BlockSpec((_FB, _D), lambda i: (i, 0)),
    out_shape=jax.ShapeDtypeStruct((_N, _D), _f32),
)


_mesh = plsc.VectorSubcoreMesh(core_axis_name="c", subcore_axis_name="s")


@functools.partial(
    pl.kernel,
    mesh=_mesh,
    compiler_params=pltpu.CompilerParams(
        needs_layout_passes=False, use_tc_tiling_on_sc=False
    ),
    out_type=[
        jax.ShapeDtypeStruct((2, _NPAD, _DT), _f32),
    ],
    scratch_types=[
        pltpu.VMEM((16,), _f32),
        [pltpu.VMEM((_K,), jnp.int32) for _ in range(3)],
        [pltpu.VMEM((_K,), jnp.int32) for _ in range(3)],
        [pltpu.VMEM((_K, _DT), _f32) for _ in range(3)],
        [pltpu.VMEM((_K, 16), _f32) for _ in range(3)],
        pltpu.VMEM((_K,), _f32),
        pltpu.VMEM_SHARED((_NPAD, _DT), _f32),
        [pltpu.SemaphoreType.DMA for _ in range(3)],
        [pltpu.SemaphoreType.DMA for _ in range(3)],
    ],
)
def _sc_edge(src_hbm, dst_hbm, tail_hbm, t_hbm, d_hbm, g_hbm, acc_hbm,
             g_v, src_s, dst_s, rows_s, dd_s, ex_v, acc_s, semg, sems):
    c = lax.axis_index("c")
    s = lax.axis_index("s")
    wid = s * 2 + c

    zero16 = jnp.zeros((16,), _f32)

    def _zrow(r, carry):
        for j in range(_DT // 16):
            rows_s[0][r, pl.ds(j * 16, 16)] = zero16
        return carry

    lax.fori_loop(0, _K, _zrow, 0)

    for t in range(_RPT // _K):
        pltpu.sync_copy(rows_s[0], acc_s.at[pl.ds(s * _RPT + t * _K, _K)])
    pltpu.sync_copy(rows_s[0], acc_s.at[pl.ds(s * _RPT + _RPT - _K, _K)])

    pltpu.sync_copy(g_hbm, g_v)
    plsc.subcore_barrier()

    iota16 = lax.iota(jnp.int32, 16)
    zero16i = jnp.zeros((16,), jnp.int32)
    l_as = jnp.full((16,), _D, jnp.int32)

    def _issue(slot, b):
        off = (wid * _NB + b) * _K

        @pl.when(off < _E)
        def _real():
            pltpu.sync_copy(src_hbm.at[pl.ds(off, _K)], src_s[slot])
            pltpu.sync_copy(dst_hbm.at[pl.ds(off, _K)], dst_s[slot])

        @pl.when(off >= _E)
        def _tail():
            pltpu.sync_copy(tail_hbm.at[pl.ds(off - _E, _K)], src_s[slot])
            pltpu.sync_copy(tail_hbm.at[pl.ds(off - _E, _K)], dst_s[slot])

        pltpu.async_copy(t_hbm.at[src_s[slot]], rows_s[slot], semg[slot])
        pltpu.async_copy(d_hbm.at[dst_s[slot]], dd_s[slot], semg[slot])

    def _wait_gather(slot):
        pltpu.make_async_copy(t_hbm.at[src_s[slot]], rows_s[slot],
                              semg[slot]).wait()
        pltpu.make_async_copy(d_hbm.at[dst_s[slot]], dd_s[slot],
                              semg[slot]).wait()

    def _wait_scatter(slot):
        pltpu.make_async_copy(rows_s[slot], acc_s.at[dst_s[slot]],
                              sems[slot]).wait()

    _issue(0, 0)
    _issue(1, 1)

    def _group(g, carry):
        for q in range(3):
            b = 3 * g + q
            rows_v = rows_s[q]
            dd_v = dd_s[q]
            _wait_gather(q)
            gv = g_v[...]
            for i in range(_K // 16):
                e16 = i * 16 + iota16
                a_s = plsc.load_gather(rows_v, [e16, l_as])
                a_d = plsc.load_gather(dd_v, [e16, zero16i])
                t0 = a_s + a_d
                al = jnp.maximum(t0, 0.2 * t0)
                s0 = gv + a_d
                sd = jnp.maximum(s0, 0.2 * s0)
                ex_v[pl.ds(i * 16, 16)] = jnp.exp(al - sd)

            @plsc.parallel_loop(0, _K, 1, unroll=16)
            def _scale(r):
                ev = plsc.load_gather(ex_v, [jnp.full((16,), r, jnp.int32)])
                for j in range(_DT // 16):
                    rows_v[r, pl.ds(j * 16, 16)] = (
                        rows_v[r, pl.ds(j * 16, 16)] * ev
                    )
            pltpu.async_copy(rows_v, acc_s.at[dst_s[q]], sems[q], add=True)

            nxt = (q + 2) % 3

            @pl.when(b + 2 < _NB)
            def _prefetch():
                @pl.when(b >= 1)
                def _drain():
                    _wait_scatter(nxt)

                _issue(nxt, b + 2)

        return carry

    lax.fori_loop(0, _NB // 3, _group, 0)
    for q in range(3):
        _wait_scatter(q)
    plsc.subcore_barrier()

    offs = [s * _RPT + t * _K for t in range(_RPT // _K)]
    offs.append(s * _RPT + _RPT - _K)
    for i, r0 in enumerate(offs):
        pltpu.async_copy(acc_s.at[pl.ds(r0, _K)], acc_hbm.at[c, pl.ds(r0, _K)],
                         semg[i % 3])
    for i, r0 in enumerate(offs):
        pltpu.make_async_copy(acc_s.at[pl.ds(r0, _K)],
                              acc_hbm.at[c, pl.ds(r0, _K)], semg[i % 3]).wait()


def kernel(x, edge_index, W1, a_src1, a_dst1, b1,
           W2, a_src2, a_dst2, b2, W3, a_src3, a_dst3, b3):
    x_pad = jnp.pad(x, ((0, _NPAD - _N), (0, 0)))
    src = edge_index[0].astype(jnp.int32)
    dst = edge_index[1].astype(jnp.int32)
    tail = jnp.concatenate([
        jnp.arange(_N, dtype=jnp.int32),
        jnp.full((_EPAD - _E - _N,), _N, jnp.int32),
    ])

    def col(v):
        return v.reshape(_D, 1)

    def row(v):
        return v.reshape(1, _D)

    t, d, g = _tc_project1(x_pad, W1, col(a_src1), col(a_dst1))
    (acc,) = _sc_edge(src, dst, tail, t, d, g[0, :16])
    t, d, g = _tc_project2(acc, row(b1), W2, col(a_src2), col(a_dst2))
    (acc,) = _sc_edge(src, dst, tail, t, d, g[0, :16])
    t, d, g = _tc_project2(acc, row(b2), W3, col(a_src3), col(a_dst3))
    (acc,) = _sc_edge(src, dst, tail, t, d, g[0, :16])
    return _tc_final(acc, row(b3))
